# baseline (speedup 1.0000x reference)
import functools

import jax
import jax.numpy as jnp
from jax import lax
from jax.experimental import pallas as pl
from jax.experimental.pallas import tpu as pltpu
from jax.experimental.pallas import tpu_sc as plsc

N = 20000
K = 2000
KPAD = 2048
OFF = 16
MASKLEN = 4128
NG = N // 16


def _greedy_body(idx_hbm, st_hbm, en_hbm, sc_hbm, cap_hbm,
                 sel_idx_hbm, sel_s_hbm, sel_e_hbm, sel_sc_hbm,
                 sel_key_hbm, cnt_hbm,
                 idx_v, st_v, en_v, sc_v, mask_v,
                 sel_idx_v, sel_s_v, sel_e_v, sel_sc_v, sel_key_v, cap_v):
    wid = lax.axis_index("s") * 2 + lax.axis_index("c")

    pltpu.sync_copy(idx_hbm, idx_v)
    pltpu.sync_copy(st_hbm, st_v)
    pltpu.sync_copy(en_hbm, en_v)
    pltpu.sync_copy(sc_hbm, sc_v)
    pltpu.sync_copy(cap_hbm, cap_v)

    lane = lax.broadcasted_iota(jnp.int32, (16,), 0)
    zero16 = jnp.zeros((16,), jnp.int32)
    big16 = jnp.full((16,), jnp.int32(2**30))

    def _zmask(i, _):
        mask_v[pl.ds(i * 16, 16)] = zero16
        return 0
    lax.fori_loop(0, MASKLEN // 16, _zmask, 0)

    def _zsel(i, _):
        sel_idx_v[pl.ds(i * 16, 16)] = zero16
        return 0
    lax.fori_loop(0, KPAD // 16, _zsel, 0)

    cap16 = cap_v[...]
    cap_s = cap16[0]

    def any16(x):
        return plsc.all_reduce_population_count(x) > 0

    def group(g, count16):
        cvec = idx_v[pl.ds(g * 16, 16)]
        svec = plsc.load_gather(st_v, [cvec])
        evec = plsc.load_gather(en_v, [cvec])
        for j in range(16):
            j16 = jnp.full((16,), jnp.int32(j))
            s16 = svec.at[j16].get(mode="promise_in_bounds")
            e16 = evec.at[j16].get(mode="promise_in_bounds")
            w16 = e16 - s16
            q = ((s16 - 9 + OFF) >> 1) + lane
            m = plsc.load_gather(mask_v, [q])
            pe = 2 * q - OFF
            po = pe + 1
            fe = m & 0x3FF
            fo = (m >> 10) & 0x3FF

            def crossf(p, f):
                c1 = (p > s16) & (p <= e16) & (
                    (f >> jnp.clip(e16 - p + 1, 0, 31)) != 0)
                c2 = (p < s16) & (
                    ((f >> jnp.clip(s16 - p, 0, 31)) & ((1 << w16) - 1)) != 0)
                return c1 | c2

            crossb = crossf(pe, fe) | crossf(po, fo)
            dupb = ((pe == s16) & (((fe >> w16) & 1) != 0)) | (
                (po == s16) & (((fo >> w16) & 1) != 0))
            cross16 = any16(crossb)
            dup16 = any16(dupb)
            accept16 = (~cross16) & (count16 < cap16)
            wmask = (lane == jnp.int32(j)) & accept16
            plsc.store_scatter(sel_idx_v, [count16], cvec, mask=wmask)
            plsc.addupdate_scatter(
                mask_v, [(s16 + OFF) >> 1],
                1 << (10 * (s16 & 1) + w16),
                mask=wmask & (~dup16))
            count16 = count16 + accept16.astype(jnp.int32)
        return count16

    def body(g, count16):
        return lax.cond(count16[0] < cap_s,
                        lambda c: group(g, c), lambda c: c, count16)

    count16 = lax.fori_loop(0, NG, body, zero16)

    def post(i, _):
        sl = pl.ds(i * 16, 16)
        iv = sel_idx_v[sl]
        sv = plsc.load_gather(st_v, [iv])
        ev = plsc.load_gather(en_v, [iv])
        scv = plsc.load_gather(sc_v, [iv])
        sel_s_v[sl] = sv
        sel_e_v[sl] = ev
        sel_sc_v[sl] = scv
        slotv = jnp.int32(i) * 16 + lane
        key = ((sv * 16 + (ev - sv)) << 11) | slotv
        sel_key_v[sl] = jnp.where(slotv < count16, key, big16)
        return 0
    lax.fori_loop(0, KPAD // 16, post, 0)

    cap_v[...] = count16

    @pl.when(wid == 0)
    def _():
        pltpu.sync_copy(sel_idx_v, sel_idx_hbm)
        pltpu.sync_copy(sel_s_v, sel_s_hbm)
        pltpu.sync_copy(sel_e_v, sel_e_hbm)
        pltpu.sync_copy(sel_sc_v, sel_sc_hbm)
        pltpu.sync_copy(sel_key_v, sel_key_hbm)
        pltpu.sync_copy(cap_v, cnt_hbm)


@jax.jit
def _greedy(idx_sorted, starts, ends, scores, cap):
    f = pl.kernel(
        _greedy_body,
        out_type=[
            jax.ShapeDtypeStruct((KPAD,), jnp.int32),
            jax.ShapeDtypeStruct((KPAD,), jnp.int32),
            jax.ShapeDtypeStruct((KPAD,), jnp.int32),
            jax.ShapeDtypeStruct((KPAD,), jnp.float32),
            jax.ShapeDtypeStruct((KPAD,), jnp.int32),
            jax.ShapeDtypeStruct((16,), jnp.int32),
        ],
        mesh=plsc.VectorSubcoreMesh(core_axis_name="c", subcore_axis_name="s"),
        compiler_params=pltpu.CompilerParams(needs_layout_passes=False),
        scratch_types=[
            pltpu.VMEM((N,), jnp.int32),
            pltpu.VMEM((N,), jnp.int32),
            pltpu.VMEM((N,), jnp.int32),
            pltpu.VMEM((N,), jnp.float32),
            pltpu.VMEM((MASKLEN,), jnp.int32),
            pltpu.VMEM((KPAD,), jnp.int32),
            pltpu.VMEM((KPAD,), jnp.int32),
            pltpu.VMEM((KPAD,), jnp.int32),
            pltpu.VMEM((KPAD,), jnp.float32),
            pltpu.VMEM((KPAD,), jnp.int32),
            pltpu.VMEM((16,), jnp.int32),
        ],
    )
    return f(idx_sorted, starts, ends, scores, cap)


def kernel(candidate_starts, candidate_ends, candidate_mention_scores,
           num_top_spans):
    starts = candidate_starts.astype(jnp.int32)
    ends = candidate_ends.astype(jnp.int32)
    scores = candidate_mention_scores.astype(jnp.float32)

    idx_sorted = jnp.argsort(-scores).astype(jnp.int32)
    cap = jnp.full((16,), jnp.minimum(num_top_spans, K), jnp.int32)

    sel_idx, sel_s, sel_e, sel_sc, sel_key, cnt = _greedy(
        idx_sorted, starts, ends, scores, cap)

    count = cnt[0]
    slot = jnp.arange(K, dtype=jnp.int32)
    order = jnp.argsort(sel_key[:K])
    idx_o = sel_idx[:K][order]
    s_o = sel_s[:K][order]
    e_o = sel_e[:K][order]
    sc_o = sel_sc[:K][order]
    occ = slot < count
    top_idx = jnp.where(occ, idx_o, idx_o[0])
    top_s = jnp.where(occ, s_o, s_o[0])
    top_e = jnp.where(occ, e_o, e_o[0])
    top_sc = jnp.where(occ, sc_o, sc_o[0])
    return top_idx, top_s, top_e, top_sc


# speedup vs baseline: 1.1409x; 1.1409x over previous
import functools

import jax
import jax.numpy as jnp
from jax import lax
from jax.experimental import pallas as pl
from jax.experimental.pallas import tpu as pltpu
from jax.experimental.pallas import tpu_sc as plsc

N = 20000
K = 2000
KPAD = 2048
OFF = 16
MASKLEN = 8240
NG = N // 16


def _greedy_body(idx_hbm, st_hbm, en_hbm, sc_hbm, cap_hbm,
                 sel_idx_hbm, sel_s_hbm, sel_e_hbm, sel_sc_hbm,
                 sel_key_hbm, cnt_hbm,
                 idx_v, st_v, en_v, sc_v, mask_v,
                 sel_idx_v, sel_s_v, sel_e_v, sel_sc_v, sel_key_v, cap_v):
    wid = lax.axis_index("s") * 2 + lax.axis_index("c")

    @pl.when(wid == 0)
    def _run():
        pltpu.sync_copy(idx_hbm, idx_v)
        pltpu.sync_copy(st_hbm, st_v)
        pltpu.sync_copy(en_hbm, en_v)
        pltpu.sync_copy(sc_hbm, sc_v)
        pltpu.sync_copy(cap_hbm, cap_v)

        lane = lax.broadcasted_iota(jnp.int32, (16,), 0)
        zero16 = jnp.zeros((16,), jnp.int32)
        big16 = jnp.full((16,), jnp.int32(2**30))

        def _zmask(i, _):
            mask_v[pl.ds(i * 16, 16)] = zero16
            return 0
        lax.fori_loop(0, MASKLEN // 16, _zmask, 0)

        def _zsel(i, _):
            sel_idx_v[pl.ds(i * 16, 16)] = zero16
            return 0
        lax.fori_loop(0, KPAD // 16, _zsel, 0)

        cap16 = cap_v[...]
        cap_s = cap16[0]

        def any16(x):
            return plsc.all_reduce_population_count(x) > 0

        def group(g, count16):
            cvec = idx_v[pl.ds(g * 16, 16)]
            svec = plsc.load_gather(st_v, [cvec])
            evec = plsc.load_gather(en_v, [cvec])
            s_sp = [svec.at[jnp.full((16,), jnp.int32(j))].get(
                mode="promise_in_bounds") for j in range(16)]
            e_sp = [evec.at[jnp.full((16,), jnp.int32(j))].get(
                mode="promise_in_bounds") for j in range(16)]

            def fetch(j):
                p0 = (s_sp[j] - 9) + lane
                p1 = (s_sp[j] + 7) + lane
                return (p0, p1,
                        plsc.load_gather(mask_v, [p0 + OFF]),
                        plsc.load_gather(mask_v, [p1 + OFF]))

            wins = [fetch(0), fetch(1)]
            preds = []
            for j in range(16):
                p0, p1, m0, m1 = wins[j]
                if j + 2 < 16:
                    wins.append(fetch(j + 2))
                s16 = s_sp[j]
                e16 = e_sp[j]
                w16 = e16 - s16

                def crossv(p, m):
                    c1 = (p > s16) & (p <= e16) & (
                        (m >> jnp.clip(e16 - p + 1, 0, 31)) != 0)
                    c2 = (p < s16) & (
                        ((m >> jnp.clip(s16 - p, 0, 31))
                         & ((1 << w16) - 1)) != 0)
                    return c1 | c2

                cross16 = any16(crossv(p0, m0) | crossv(p1, m1))
                dup16 = any16((p0 == s16) & (((m0 >> w16) & 1) != 0))
                for (sp, ep, ap) in preds[-2:]:
                    pc = ap & (((s16 < sp) & (sp <= e16) & (ep > e16)) |
                               ((sp < s16) & (s16 <= ep) & (ep < e16)))
                    cross16 = cross16 | pc
                    dup16 = dup16 | (ap & (sp == s16) & (ep == e16))
                accept16 = (~cross16) & (count16 < cap16)
                wmask = (lane == jnp.int32(j)) & accept16
                plsc.store_scatter(sel_idx_v, [count16], cvec, mask=wmask)
                plsc.addupdate_scatter(
                    mask_v, [s16 + OFF], 1 << w16, mask=wmask & (~dup16))
                count16 = count16 + accept16.astype(jnp.int32)
                preds.append((s16, e16, accept16))
            return count16

        def body(g, count16):
            return lax.cond(count16[0] < cap_s,
                            lambda c: group(g, c), lambda c: c, count16)

        count16 = lax.fori_loop(0, NG, body, zero16)

        def post(i, _):
            sl = pl.ds(i * 16, 16)
            iv = sel_idx_v[sl]
            sv = plsc.load_gather(st_v, [iv])
            ev = plsc.load_gather(en_v, [iv])
            scv = plsc.load_gather(sc_v, [iv])
            sel_s_v[sl] = sv
            sel_e_v[sl] = ev
            sel_sc_v[sl] = scv
            slotv = jnp.int32(i) * 16 + lane
            key = ((sv * 16 + (ev - sv)) << 11) | slotv
            sel_key_v[sl] = jnp.where(slotv < count16, key, big16)
            return 0
        lax.fori_loop(0, KPAD // 16, post, 0)

        cap_v[...] = count16

        pltpu.sync_copy(sel_idx_v, sel_idx_hbm)
        pltpu.sync_copy(sel_s_v, sel_s_hbm)
        pltpu.sync_copy(sel_e_v, sel_e_hbm)
        pltpu.sync_copy(sel_sc_v, sel_sc_hbm)
        pltpu.sync_copy(sel_key_v, sel_key_hbm)
        pltpu.sync_copy(cap_v, cnt_hbm)


@jax.jit
def _greedy(idx_sorted, starts, ends, scores, cap):
    f = pl.kernel(
        _greedy_body,
        out_type=[
            jax.ShapeDtypeStruct((KPAD,), jnp.int32),
            jax.ShapeDtypeStruct((KPAD,), jnp.int32),
            jax.ShapeDtypeStruct((KPAD,), jnp.int32),
            jax.ShapeDtypeStruct((KPAD,), jnp.float32),
            jax.ShapeDtypeStruct((KPAD,), jnp.int32),
            jax.ShapeDtypeStruct((16,), jnp.int32),
        ],
        mesh=plsc.VectorSubcoreMesh(core_axis_name="c", subcore_axis_name="s"),
        compiler_params=pltpu.CompilerParams(needs_layout_passes=False),
        scratch_types=[
            pltpu.VMEM((N,), jnp.int32),
            pltpu.VMEM((N,), jnp.int32),
            pltpu.VMEM((N,), jnp.int32),
            pltpu.VMEM((N,), jnp.float32),
            pltpu.VMEM((MASKLEN,), jnp.int32),
            pltpu.VMEM((KPAD,), jnp.int32),
            pltpu.VMEM((KPAD,), jnp.int32),
            pltpu.VMEM((KPAD,), jnp.int32),
            pltpu.VMEM((KPAD,), jnp.float32),
            pltpu.VMEM((KPAD,), jnp.int32),
            pltpu.VMEM((16,), jnp.int32),
        ],
    )
    return f(idx_sorted, starts, ends, scores, cap)


def kernel(candidate_starts, candidate_ends, candidate_mention_scores,
           num_top_spans):
    starts = candidate_starts.astype(jnp.int32)
    ends = candidate_ends.astype(jnp.int32)
    scores = candidate_mention_scores.astype(jnp.float32)

    idx_sorted = jnp.argsort(-scores).astype(jnp.int32)
    cap = jnp.full((16,), jnp.minimum(num_top_spans, K), jnp.int32)

    sel_idx, sel_s, sel_e, sel_sc, sel_key, cnt = _greedy(
        idx_sorted, starts, ends, scores, cap)

    count = cnt[0]
    slot = jnp.arange(K, dtype=jnp.int32)
    _, idx_o, s_o, e_o, sc_o = lax.sort(
        (sel_key[:K], sel_idx[:K], sel_s[:K], sel_e[:K], sel_sc[:K]),
        dimension=0, is_stable=False, num_keys=1)
    occ = slot < count
    top_idx = jnp.where(occ, idx_o, idx_o[0])
    top_s = jnp.where(occ, s_o, s_o[0])
    top_e = jnp.where(occ, e_o, e_o[0])
    top_sc = jnp.where(occ, sc_o, sc_o[0])
    return top_idx, top_s, top_e, top_sc


# speedup vs baseline: 1.6264x; 1.4255x over previous
import functools

import jax
import jax.numpy as jnp
from jax import lax
from jax.experimental import pallas as pl
from jax.experimental.pallas import tpu as pltpu
from jax.experimental.pallas import tpu_sc as plsc

N = 20000
K = 2000
KPAD = 2048
OFF = 16
MASKLEN = 8240
NG = N // 16


def _greedy_body(idx_hbm, st_hbm, en_hbm, sc_hbm, cap_hbm,
                 sel_idx_hbm, sel_s_hbm, sel_e_hbm, sel_sc_hbm,
                 sel_key_hbm, cnt_hbm,
                 idx_v, st_v, en_v, sc_v, mask_v,
                 sel_idx_v, sel_s_v, sel_e_v, sel_sc_v, sel_key_v, cap_v,
                 sem):
    wid = lax.axis_index("s") * 2 + lax.axis_index("c")

    @pl.when(wid == 0)
    def _run():
        cps = [pltpu.async_copy(s, d, sem) for s, d in (
            (idx_hbm, idx_v), (st_hbm, st_v), (en_hbm, en_v),
            (sc_hbm, sc_v), (cap_hbm, cap_v))]

        lane = lax.broadcasted_iota(jnp.int32, (16,), 0)
        zero16 = jnp.zeros((16,), jnp.int32)
        big16 = jnp.full((16,), jnp.int32(2**30))

        def _zmask(i, _):
            mask_v[pl.ds(i * 16, 16)] = zero16
            return 0
        lax.fori_loop(0, MASKLEN // 16, _zmask, 0)

        def _zsel(i, _):
            sel_idx_v[pl.ds(i * 16, 16)] = zero16
            return 0
        lax.fori_loop(0, KPAD // 16, _zsel, 0)

        for cp in cps:
            cp.wait()

        cap16 = cap_v[...]
        cap_s = cap16[0]

        def any16(x):
            return plsc.all_reduce_population_count(x) > 0

        gt9 = lane > 9
        sh2c = jnp.where(lane == 9, 31, (9 - lane) & 31)
        k0 = 9 - lane
        k1 = (-7) - lane
        nine16 = jnp.full((16,), jnp.int32(9))

        def group(g, count16):
            cvec = idx_v[pl.ds(g * 16, 16)]
            svec = plsc.load_gather(st_v, [cvec])
            evec = plsc.load_gather(en_v, [cvec])
            pk = (svec << 4) | (evec - svec)

            def splat(j):
                pk16 = pk.at[jnp.full((16,), jnp.int32(j))].get(
                    mode="promise_in_bounds")
                return pk16 >> 4, pk16 & 15

            def fetch(s16):
                p0 = (s16 - 9) + lane
                p1 = (s16 + 7) + lane
                return (plsc.load_gather(mask_v, [p0 + OFF]),
                        plsc.load_gather(mask_v, [p1 + OFF]))

            sw = [splat(0)]
            wins = [fetch(sw[0][0])]
            pred = None
            for j in range(16):
                m0, m1 = wins[j]
                s16, w16 = sw[j]
                if j + 1 < 16:
                    sw.append(splat(j + 1))
                    wins.append(fetch(sw[j + 1][0]))
                e16 = s16 + w16
                wm16 = (1 << w16) - 1
                c10 = gt9 & (((m0 >> 1) >> ((w16 + k0) & 31)) != 0)
                c20 = ((m0 >> sh2c) & wm16) != 0
                c11 = ((m1 >> 1) >> ((w16 + k1) & 31)) != 0
                cross16 = any16(c10 | c20 | c11)
                dup16 = ((m0.at[nine16].get(mode="promise_in_bounds")
                          >> w16) & 1) != 0
                if pred is not None:
                    sp, ep, ap = pred
                    ssp = s16 < sp
                    sps = sp < s16
                    eep = e16 < ep
                    epe = ep < e16
                    ovl = (sp <= e16) & (s16 <= ep)
                    cross16 = cross16 | (ap & ovl & ((ssp & eep) |
                                                     (sps & epe)))
                    dup16 = dup16 | (ap & (~ssp) & (~sps) &
                                     (~eep) & (~epe))
                accept16 = (~cross16) & (count16 < cap16)
                wmask = (lane == jnp.int32(j)) & accept16
                plsc.store_scatter(sel_idx_v, [count16], cvec, mask=wmask)
                plsc.addupdate_scatter(
                    mask_v, [s16 + OFF], 1 << w16, mask=wmask & (~dup16))
                count16 = count16 + accept16.astype(jnp.int32)
                pred = (s16, e16, accept16)
            return count16

        def body(g, count16):
            return lax.cond(count16[0] < cap_s,
                            lambda c: group(g, c), lambda c: c, count16)

        def block(b, count16):
            def run(c):
                return lax.fori_loop(b * 25, (b + 1) * 25, body, c)
            return lax.cond(count16[0] < cap_s, run, lambda c: c, count16)

        count16 = lax.fori_loop(0, NG // 25, block, zero16)

        def post(i, _):
            sl = pl.ds(i * 16, 16)
            iv = sel_idx_v[sl]
            sv = plsc.load_gather(st_v, [iv])
            ev = plsc.load_gather(en_v, [iv])
            scv = plsc.load_gather(sc_v, [iv])
            sel_s_v[sl] = sv
            sel_e_v[sl] = ev
            sel_sc_v[sl] = scv
            slotv = jnp.int32(i) * 16 + lane
            key = ((sv * 16 + (ev - sv)) << 11) | slotv
            sel_key_v[sl] = jnp.where(slotv < count16, key, big16)
            return 0
        lax.fori_loop(0, KPAD // 16, post, 0)

        cap_v[...] = count16

        pltpu.sync_copy(sel_idx_v, sel_idx_hbm)
        pltpu.sync_copy(sel_s_v, sel_s_hbm)
        pltpu.sync_copy(sel_e_v, sel_e_hbm)
        pltpu.sync_copy(sel_sc_v, sel_sc_hbm)
        pltpu.sync_copy(sel_key_v, sel_key_hbm)
        pltpu.sync_copy(cap_v, cnt_hbm)


@jax.jit
def _greedy(idx_sorted, starts, ends, scores, cap):
    f = pl.kernel(
        _greedy_body,
        out_type=[
            jax.ShapeDtypeStruct((KPAD,), jnp.int32),
            jax.ShapeDtypeStruct((KPAD,), jnp.int32),
            jax.ShapeDtypeStruct((KPAD,), jnp.int32),
            jax.ShapeDtypeStruct((KPAD,), jnp.float32),
            jax.ShapeDtypeStruct((KPAD,), jnp.int32),
            jax.ShapeDtypeStruct((16,), jnp.int32),
        ],
        mesh=plsc.VectorSubcoreMesh(core_axis_name="c", subcore_axis_name="s"),
        compiler_params=pltpu.CompilerParams(needs_layout_passes=False),
        scratch_types=[
            pltpu.VMEM((N,), jnp.int32),
            pltpu.VMEM((N,), jnp.int32),
            pltpu.VMEM((N,), jnp.int32),
            pltpu.VMEM((N,), jnp.float32),
            pltpu.VMEM((MASKLEN,), jnp.int32),
            pltpu.VMEM((KPAD,), jnp.int32),
            pltpu.VMEM((KPAD,), jnp.int32),
            pltpu.VMEM((KPAD,), jnp.int32),
            pltpu.VMEM((KPAD,), jnp.float32),
            pltpu.VMEM((KPAD,), jnp.int32),
            pltpu.VMEM((16,), jnp.int32),
            pltpu.SemaphoreType.DMA,
        ],
    )
    return f(idx_sorted, starts, ends, scores, cap)


def kernel(candidate_starts, candidate_ends, candidate_mention_scores,
           num_top_spans):
    starts = candidate_starts.astype(jnp.int32)
    ends = candidate_ends.astype(jnp.int32)
    scores = candidate_mention_scores.astype(jnp.float32)

    idx_sorted = jnp.argsort(-scores).astype(jnp.int32)
    cap = jnp.full((16,), jnp.minimum(num_top_spans, K), jnp.int32)

    sel_idx, sel_s, sel_e, sel_sc, sel_key, cnt = _greedy(
        idx_sorted, starts, ends, scores, cap)

    count = cnt[0]
    slot = jnp.arange(K, dtype=jnp.int32)
    _, idx_o, s_o, e_o, sc_o = lax.sort(
        (sel_key[:K], sel_idx[:K], sel_s[:K], sel_e[:K], sel_sc[:K]),
        dimension=0, is_stable=False, num_keys=1)
    occ = slot < count
    top_idx = jnp.where(occ, idx_o, idx_o[0])
    top_s = jnp.where(occ, s_o, s_o[0])
    top_e = jnp.where(occ, e_o, e_o[0])
    top_sc = jnp.where(occ, sc_o, sc_o[0])
    return top_idx, top_s, top_e, top_sc


# speedup vs baseline: 1.6609x; 1.0212x over previous
import functools

import jax
import jax.numpy as jnp
from jax import lax
from jax.experimental import pallas as pl
from jax.experimental.pallas import tpu as pltpu
from jax.experimental.pallas import tpu_sc as plsc

N = 20000
K = 2000
KPAD = 2048
OFF = 16
MASKLEN = 8240
NG = N // 16


def _greedy_body(idx_hbm, st_hbm, en_hbm, sc_hbm, cap_hbm,
                 sel_idx_hbm, sel_s_hbm, sel_e_hbm, sel_sc_hbm,
                 sel_key_hbm, cnt_hbm,
                 idx_v, st_v, en_v, sc_v, mask_v,
                 sel_idx_v, sel_s_v, sel_e_v, sel_sc_v, sel_key_v, cap_v,
                 sem):
    wid = lax.axis_index("s") * 2 + lax.axis_index("c")

    @pl.when(wid == 0)
    def _run():
        cps = [pltpu.async_copy(s, d, sem) for s, d in (
            (idx_hbm, idx_v), (st_hbm, st_v), (en_hbm, en_v),
            (sc_hbm, sc_v), (cap_hbm, cap_v))]

        lane = lax.broadcasted_iota(jnp.int32, (16,), 0)
        zero16 = jnp.zeros((16,), jnp.int32)
        big16 = jnp.full((16,), jnp.int32(2**30))

        def _zmask(i, _):
            for u in range(5):
                mask_v[pl.ds(i * 80 + u * 16, 16)] = zero16
            return 0
        lax.fori_loop(0, MASKLEN // 80, _zmask, 0)

        def _zsel(i, _):
            for u in range(4):
                sel_idx_v[pl.ds(i * 64 + u * 16, 16)] = zero16
            return 0
        lax.fori_loop(0, KPAD // 64, _zsel, 0)

        for cp in cps:
            cp.wait()

        cap16 = cap_v[...]
        cap_s = cap16[0]

        def any16(x):
            return plsc.all_reduce_population_count(x) > 0

        gt9i = jnp.where(lane > 9, -1, 0)
        sh2c = jnp.where(lane == 9, 31, (9 - lane) & 31)
        k0 = 9 - lane
        k1 = (-7) - lane
        nine16 = jnp.full((16,), jnp.int32(9))

        def group(g, count16):
            cvec = idx_v[pl.ds(g * 16, 16)]
            svec = plsc.load_gather(st_v, [cvec])
            evec = plsc.load_gather(en_v, [cvec])
            pk = (svec << 4) | (evec - svec)

            def splat(j):
                pk16 = pk.at[jnp.full((16,), jnp.int32(j))].get(
                    mode="promise_in_bounds")
                return pk16 >> 4, pk16 & 15, pk16

            def fetch(s16):
                p0 = (s16 - 9) + lane
                p1 = (s16 + 7) + lane
                return (plsc.load_gather(mask_v, [p0 + OFF]),
                        plsc.load_gather(mask_v, [p1 + OFF]))

            sw = [splat(0)]
            wins = [fetch(sw[0][0])]
            pred = None
            for j in range(16):
                m0, m1 = wins[j]
                s16, w16, pk16 = sw[j]
                if j + 1 < 16:
                    sw.append(splat(j + 1))
                    wins.append(fetch(sw[j + 1][0]))
                e16 = s16 + w16
                bit16 = 1 << w16
                wm16 = bit16 - 1
                f10 = ((m0 >> 1) >> ((w16 + k0) & 31)) & gt9i
                f20 = (m0 >> sh2c) & wm16
                f11 = (m1 >> 1) >> ((w16 + k1) & 31)
                cross16 = any16((f10 | f20 | f11) != 0)
                dup16 = ((m0.at[nine16].get(mode="promise_in_bounds")
                          >> w16) & 1) != 0
                if pred is not None:
                    sp, ep, ap, pkp = pred
                    cross16 = cross16 | (ap & (
                        ((s16 < sp) & (sp <= e16) & (e16 < ep)) |
                        ((sp < s16) & (s16 <= ep) & (ep < e16))))
                    dup16 = dup16 | (ap & (pkp == pk16))
                accept16 = (~cross16) & (count16 < cap16)
                wmask = (lane == jnp.int32(j)) & accept16
                plsc.store_scatter(sel_idx_v, [count16], cvec, mask=wmask)
                plsc.addupdate_scatter(
                    mask_v, [s16 + OFF], bit16, mask=wmask & (~dup16))
                count16 = count16 + accept16.astype(jnp.int32)
                pred = (s16, e16, accept16, pk16)
            return count16

        def body(g, count16):
            return lax.cond(count16[0] < cap_s,
                            lambda c: group(g, c), lambda c: c, count16)

        def block(b, count16):
            def run(c):
                return lax.fori_loop(b * 25, (b + 1) * 25, body, c)
            return lax.cond(count16[0] < cap_s, run, lambda c: c, count16)

        count16 = lax.fori_loop(0, NG // 25, block, zero16)

        def post(i, _):
            sl = pl.ds(i * 16, 16)
            iv = sel_idx_v[sl]
            sv = plsc.load_gather(st_v, [iv])
            ev = plsc.load_gather(en_v, [iv])
            scv = plsc.load_gather(sc_v, [iv])
            sel_s_v[sl] = sv
            sel_e_v[sl] = ev
            sel_sc_v[sl] = scv
            slotv = jnp.int32(i) * 16 + lane
            key = ((sv * 16 + (ev - sv)) << 11) | slotv
            sel_key_v[sl] = jnp.where(slotv < count16, key, big16)
            return 0
        lax.fori_loop(0, KPAD // 16, post, 0)

        cap_v[...] = count16

        pltpu.sync_copy(sel_idx_v, sel_idx_hbm)
        pltpu.sync_copy(sel_s_v, sel_s_hbm)
        pltpu.sync_copy(sel_e_v, sel_e_hbm)
        pltpu.sync_copy(sel_sc_v, sel_sc_hbm)
        pltpu.sync_copy(sel_key_v, sel_key_hbm)
        pltpu.sync_copy(cap_v, cnt_hbm)


@jax.jit
def _greedy(idx_sorted, starts, ends, scores, cap):
    f = pl.kernel(
        _greedy_body,
        out_type=[
            jax.ShapeDtypeStruct((KPAD,), jnp.int32),
            jax.ShapeDtypeStruct((KPAD,), jnp.int32),
            jax.ShapeDtypeStruct((KPAD,), jnp.int32),
            jax.ShapeDtypeStruct((KPAD,), jnp.float32),
            jax.ShapeDtypeStruct((KPAD,), jnp.int32),
            jax.ShapeDtypeStruct((16,), jnp.int32),
        ],
        mesh=plsc.VectorSubcoreMesh(core_axis_name="c", subcore_axis_name="s"),
        compiler_params=pltpu.CompilerParams(needs_layout_passes=False),
        scratch_types=[
            pltpu.VMEM((N,), jnp.int32),
            pltpu.VMEM((N,), jnp.int32),
            pltpu.VMEM((N,), jnp.int32),
            pltpu.VMEM((N,), jnp.float32),
            pltpu.VMEM((MASKLEN,), jnp.int32),
            pltpu.VMEM((KPAD,), jnp.int32),
            pltpu.VMEM((KPAD,), jnp.int32),
            pltpu.VMEM((KPAD,), jnp.int32),
            pltpu.VMEM((KPAD,), jnp.float32),
            pltpu.VMEM((KPAD,), jnp.int32),
            pltpu.VMEM((16,), jnp.int32),
            pltpu.SemaphoreType.DMA,
        ],
    )
    return f(idx_sorted, starts, ends, scores, cap)


def kernel(candidate_starts, candidate_ends, candidate_mention_scores,
           num_top_spans):
    starts = candidate_starts.astype(jnp.int32)
    ends = candidate_ends.astype(jnp.int32)
    scores = candidate_mention_scores.astype(jnp.float32)

    idx_sorted = jnp.argsort(-scores).astype(jnp.int32)
    cap = jnp.full((16,), jnp.minimum(num_top_spans, K), jnp.int32)

    sel_idx, sel_s, sel_e, sel_sc, sel_key, cnt = _greedy(
        idx_sorted, starts, ends, scores, cap)

    count = cnt[0]
    slot = jnp.arange(K, dtype=jnp.int32)
    _, idx_o, s_o, e_o, sc_o = lax.sort(
        (sel_key[:K], sel_idx[:K], sel_s[:K], sel_e[:K], sel_sc[:K]),
        dimension=0, is_stable=False, num_keys=1)
    occ = slot < count
    top_idx = jnp.where(occ, idx_o, idx_o[0])
    top_s = jnp.where(occ, s_o, s_o[0])
    top_e = jnp.where(occ, e_o, e_o[0])
    top_sc = jnp.where(occ, sc_o, sc_o[0])
    return top_idx, top_s, top_e, top_sc


# speedup vs baseline: 1.7043x; 1.0261x over previous
import functools

import jax
import jax.numpy as jnp
from jax import lax
from jax.experimental import pallas as pl
from jax.experimental.pallas import tpu as pltpu
from jax.experimental.pallas import tpu_sc as plsc

N = 20000
K = 2000
KPAD = 2048
OFF = 16
MASKLEN = 8240
NG = N // 16


def _greedy_body(idx_hbm, st_hbm, en_hbm, sc_hbm, cap_hbm,
                 sel_idx_hbm, sel_s_hbm, sel_e_hbm, sel_sc_hbm,
                 sel_key_hbm, cnt_hbm,
                 idx_v, st_v, en_v, sc_v, mask_v,
                 sel_idx_v, sel_s_v, sel_e_v, sel_sc_v, sel_key_v, cap_v,
                 sem):
    wid = lax.axis_index("s") * 2 + lax.axis_index("c")

    @pl.when(wid == 0)
    def _run():
        cps = [pltpu.async_copy(s, d, sem) for s, d in (
            (idx_hbm, idx_v), (st_hbm, st_v), (en_hbm, en_v),
            (sc_hbm, sc_v), (cap_hbm, cap_v))]

        lane = lax.broadcasted_iota(jnp.int32, (16,), 0)
        zero16 = jnp.zeros((16,), jnp.int32)
        big16 = jnp.full((16,), jnp.int32(2**30))

        def _zmask(i, _):
            for u in range(5):
                mask_v[pl.ds(i * 80 + u * 16, 16)] = zero16
            return 0
        lax.fori_loop(0, MASKLEN // 80, _zmask, 0)

        def _zsel(i, _):
            for u in range(4):
                sel_idx_v[pl.ds(i * 64 + u * 16, 16)] = zero16
            return 0
        lax.fori_loop(0, KPAD // 64, _zsel, 0)

        for cp in cps:
            cp.wait()

        cap16 = cap_v[...]
        cap_s = cap16[0]

        def any16(x):
            return plsc.all_reduce_population_count(x) > 0

        gt9i = jnp.where(lane > 9, -1, 0)
        sh2c = jnp.where(lane == 9, 31, (9 - lane) & 31)
        k0 = 9 - lane
        k1 = (-7) - lane
        lane9 = lane == 9

        def group(g, count16):
            cvec = idx_v[pl.ds(g * 16, 16)]
            svec = plsc.load_gather(st_v, [cvec])
            evec = plsc.load_gather(en_v, [cvec])
            pk = (svec << 4) | (evec - svec)

            def splat(j):
                pk16 = pk.at[jnp.full((16,), jnp.int32(j))].get(
                    mode="promise_in_bounds")
                return pk16 >> 4, pk16 & 15

            def fetch(s16):
                p0 = (s16 - 9) + lane
                p1 = (s16 + 7) + lane
                return (plsc.load_gather(mask_v, [p0 + OFF]),
                        plsc.load_gather(mask_v, [p1 + OFF]))

            sw = [splat(0)]
            wins = [fetch(sw[0][0])]
            pred = None
            for j in range(16):
                m0, m1 = wins[j]
                s16, w16 = sw[j]
                if j + 1 < 16:
                    sw.append(splat(j + 1))
                    wins.append(fetch(sw[j + 1][0]))
                e16 = s16 + w16
                bit16 = 1 << w16
                wm16 = bit16 - 1
                f10 = ((m0 >> 1) >> ((w16 + k0) & 31)) & gt9i
                f20 = (m0 >> sh2c) & wm16
                f11 = (m1 >> 1) >> ((w16 + k1) & 31)
                cross16 = any16((f10 | f20 | f11) != 0)
                mword = m0 | bit16
                if pred is not None:
                    sp, ep, ap, pb = pred
                    cross16 = cross16 | (ap & (
                        ((s16 < sp) & (sp <= e16) & (e16 < ep)) |
                        ((sp < s16) & (s16 <= ep) & (ep < e16))))
                    mword = mword | jnp.where(ap & (sp == s16), pb, 0)
                accept16 = (~cross16) & (count16 < cap16)
                wmask = (lane == jnp.int32(j)) & accept16
                plsc.store_scatter(sel_idx_v, [count16], cvec, mask=wmask)
                plsc.store_scatter(
                    mask_v, [s16 + OFF], mword, mask=lane9 & accept16)
                count16 = count16 + accept16.astype(jnp.int32)
                pred = (s16, e16, accept16, bit16)
            return count16

        def body(g, count16):
            return lax.cond(count16[0] < cap_s,
                            lambda c: group(g, c), lambda c: c, count16)

        def block(b, count16):
            def run(c):
                return lax.fori_loop(b * 25, (b + 1) * 25, body, c)
            return lax.cond(count16[0] < cap_s, run, lambda c: c, count16)

        count16 = lax.fori_loop(0, NG // 25, block, zero16)

        def post(i, _):
            sl = pl.ds(i * 16, 16)
            iv = sel_idx_v[sl]
            sv = plsc.load_gather(st_v, [iv])
            ev = plsc.load_gather(en_v, [iv])
            scv = plsc.load_gather(sc_v, [iv])
            sel_s_v[sl] = sv
            sel_e_v[sl] = ev
            sel_sc_v[sl] = scv
            slotv = jnp.int32(i) * 16 + lane
            key = ((sv * 16 + (ev - sv)) << 11) | slotv
            sel_key_v[sl] = jnp.where(slotv < count16, key, big16)
            return 0
        lax.fori_loop(0, KPAD // 16, post, 0)

        cap_v[...] = count16

        pltpu.sync_copy(sel_idx_v, sel_idx_hbm)
        pltpu.sync_copy(sel_s_v, sel_s_hbm)
        pltpu.sync_copy(sel_e_v, sel_e_hbm)
        pltpu.sync_copy(sel_sc_v, sel_sc_hbm)
        pltpu.sync_copy(sel_key_v, sel_key_hbm)
        pltpu.sync_copy(cap_v, cnt_hbm)


@jax.jit
def _greedy(idx_sorted, starts, ends, scores, cap):
    f = pl.kernel(
        _greedy_body,
        out_type=[
            jax.ShapeDtypeStruct((KPAD,), jnp.int32),
            jax.ShapeDtypeStruct((KPAD,), jnp.int32),
            jax.ShapeDtypeStruct((KPAD,), jnp.int32),
            jax.ShapeDtypeStruct((KPAD,), jnp.float32),
            jax.ShapeDtypeStruct((KPAD,), jnp.int32),
            jax.ShapeDtypeStruct((16,), jnp.int32),
        ],
        mesh=plsc.VectorSubcoreMesh(core_axis_name="c", subcore_axis_name="s"),
        compiler_params=pltpu.CompilerParams(needs_layout_passes=False),
        scratch_types=[
            pltpu.VMEM((N,), jnp.int32),
            pltpu.VMEM((N,), jnp.int32),
            pltpu.VMEM((N,), jnp.int32),
            pltpu.VMEM((N,), jnp.float32),
            pltpu.VMEM((MASKLEN,), jnp.int32),
            pltpu.VMEM((KPAD,), jnp.int32),
            pltpu.VMEM((KPAD,), jnp.int32),
            pltpu.VMEM((KPAD,), jnp.int32),
            pltpu.VMEM((KPAD,), jnp.float32),
            pltpu.VMEM((KPAD,), jnp.int32),
            pltpu.VMEM((16,), jnp.int32),
            pltpu.SemaphoreType.DMA,
        ],
    )
    return f(idx_sorted, starts, ends, scores, cap)


def kernel(candidate_starts, candidate_ends, candidate_mention_scores,
           num_top_spans):
    starts = candidate_starts.astype(jnp.int32)
    ends = candidate_ends.astype(jnp.int32)
    scores = candidate_mention_scores.astype(jnp.float32)

    idx_sorted = jnp.argsort(-scores).astype(jnp.int32)
    cap = jnp.full((16,), jnp.minimum(num_top_spans, K), jnp.int32)

    sel_idx, sel_s, sel_e, sel_sc, sel_key, cnt = _greedy(
        idx_sorted, starts, ends, scores, cap)

    count = cnt[0]
    slot = jnp.arange(K, dtype=jnp.int32)
    _, idx_o, s_o, e_o, sc_o = lax.sort(
        (sel_key[:K], sel_idx[:K], sel_s[:K], sel_e[:K], sel_sc[:K]),
        dimension=0, is_stable=False, num_keys=1)
    occ = slot < count
    top_idx = jnp.where(occ, idx_o, idx_o[0])
    top_s = jnp.where(occ, s_o, s_o[0])
    top_e = jnp.where(occ, e_o, e_o[0])
    top_sc = jnp.where(occ, sc_o, sc_o[0])
    return top_idx, top_s, top_e, top_sc
